# baseline (speedup 1.0000x reference)
import jax
import jax.numpy as jnp
from jax.experimental import pallas as pl

_RBLK = 16


def _body(w_ref, x_ref, o_ref):
    o_ref[...] = jax.lax.dot_general(
        w_ref[...], x_ref[...],
        dimension_numbers=(((1,), (0,)), ((), ())),
        preferred_element_type=jnp.float32,
    )


def kernel(superposition_weights, W):
    E, B = superposition_weights.shape
    _, d1, d2 = W.shape
    wT = superposition_weights.T
    out = pl.pallas_call(
        _body,
        grid=(d1 // _RBLK,),
        in_specs=[
            pl.BlockSpec((B, E), lambda i: (0, 0)),
            pl.BlockSpec((E, _RBLK, d2), lambda i: (0, i, 0)),
        ],
        out_specs=pl.BlockSpec((B, _RBLK, d2), lambda i: (0, i, 0)),
        out_shape=jax.ShapeDtypeStruct((B, d1, d2), jnp.float32),
    )(wT, W)
    return out


# speedup vs baseline: 1.4651x; 1.4651x over previous
import jax
import jax.numpy as jnp
from jax.experimental import pallas as pl

_RBLK = 32


def _body(w_ref, x_ref, o_ref):
    o_ref[...] = jax.lax.dot_general(
        w_ref[...], x_ref[...],
        dimension_numbers=(((0,), (0,)), ((), ())),
        preferred_element_type=jnp.float32,
    )


def kernel(superposition_weights, W):
    E, B = superposition_weights.shape
    _, d1, d2 = W.shape
    out = pl.pallas_call(
        _body,
        grid=(d1 // _RBLK,),
        in_specs=[
            pl.BlockSpec((E, B), lambda i: (0, 0)),
            pl.BlockSpec((E, _RBLK, d2), lambda i: (0, i, 0)),
        ],
        out_specs=pl.BlockSpec((B, _RBLK, d2), lambda i: (0, i, 0)),
        out_shape=jax.ShapeDtypeStruct((B, d1, d2), jnp.float32),
    )(superposition_weights, W)
    return out


# speedup vs baseline: 1.7695x; 1.2078x over previous
import jax
import jax.numpy as jnp
from jax.experimental import pallas as pl

_RBLK = 64


def _body(w_ref, x_ref, o_ref):
    o_ref[...] = jax.lax.dot_general(
        w_ref[...], x_ref[...],
        dimension_numbers=(((0,), (0,)), ((), ())),
        preferred_element_type=jnp.float32,
    )


def kernel(superposition_weights, W):
    E, B = superposition_weights.shape
    _, d1, d2 = W.shape
    out = pl.pallas_call(
        _body,
        grid=(d1 // _RBLK,),
        in_specs=[
            pl.BlockSpec((E, B), lambda i: (0, 0)),
            pl.BlockSpec((E, _RBLK, d2), lambda i: (0, i, 0)),
        ],
        out_specs=pl.BlockSpec((B, _RBLK, d2), lambda i: (0, i, 0)),
        out_shape=jax.ShapeDtypeStruct((B, d1, d2), jnp.float32),
    )(superposition_weights, W)
    return out


# speedup vs baseline: 1.7948x; 1.0143x over previous
import jax
import jax.numpy as jnp
from jax.experimental import pallas as pl

_RBLK = 128


def _body(w_ref, x_ref, o_ref):
    o_ref[...] = jax.lax.dot_general(
        w_ref[...], x_ref[...],
        dimension_numbers=(((0,), (0,)), ((), ())),
        preferred_element_type=jnp.float32,
    )


def kernel(superposition_weights, W):
    E, B = superposition_weights.shape
    _, d1, d2 = W.shape
    out = pl.pallas_call(
        _body,
        grid=(d1 // _RBLK,),
        in_specs=[
            pl.BlockSpec((E, B), lambda i: (0, 0)),
            pl.BlockSpec((E, _RBLK, d2), lambda i: (0, i, 0)),
        ],
        out_specs=pl.BlockSpec((B, _RBLK, d2), lambda i: (0, i, 0)),
        out_shape=jax.ShapeDtypeStruct((B, d1, d2), jnp.float32),
    )(superposition_weights, W)
    return out
